# baseline (speedup 1.0000x reference)
import functools

import jax
import jax.numpy as jnp
from jax import lax
from jax.experimental import pallas as pl
from jax.experimental.pallas import tpu as pltpu
from jax.experimental.pallas import tpu_sc as plsc

BLOCK_ROWS = 512
N_EXPERTS = 16
TOPK = 2
NC, NS, LANES = 2, 16, 16
NW = NC * NS


def _logits_block(x_ref, w_ref, b_ref, logits_ref):
    logits_ref[...] = jnp.dot(x_ref[...], w_ref[...],
                              preferred_element_type=jnp.float32) + b_ref[...]


def _tc_logits(x, W, b2):
    rows = x.shape[0]
    return pl.pallas_call(
        _logits_block,
        grid=(rows // BLOCK_ROWS,),
        in_specs=[
            pl.BlockSpec((BLOCK_ROWS, x.shape[1]), lambda i: (i, 0)),
            pl.BlockSpec((x.shape[1], N_EXPERTS), lambda i: (0, 0)),
            pl.BlockSpec((1, N_EXPERTS), lambda i: (0, 0)),
        ],
        out_specs=pl.BlockSpec((BLOCK_ROWS, N_EXPERTS), lambda i: (i, 0)),
        out_shape=jax.ShapeDtypeStruct((rows, N_EXPERTS), jnp.float32),
    )(x, W, b2)


def _sc_route(logits):
    rows = logits.shape[0]
    rpw = rows // NW
    groups = rpw // 8
    mesh = plsc.VectorSubcoreMesh(core_axis_name="c", subcore_axis_name="s",
                                  num_cores=NC, num_subcores=NS)

    @functools.partial(
        pl.kernel,
        out_type=[
            jax.ShapeDtypeStruct((rows, N_EXPERTS), jnp.float32),
            jax.ShapeDtypeStruct((rows * TOPK,), jnp.int32),
        ],
        mesh=mesh,
        scratch_types=[
            pltpu.VMEM((rpw, N_EXPERTS), jnp.float32),
            pltpu.VMEM((rpw * TOPK,), jnp.int32),
        ],
    )
    def route(logits_hbm, gates_hbm, idx_hbm, lg_v, idx_v):
        wid = lax.axis_index("s") * NC + lax.axis_index("c")
        base = wid * rpw
        pltpu.sync_copy(logits_hbm.at[pl.ds(base, rpw)], lg_v)
        lane = lax.iota(jnp.int32, 16)
        neg_inf = jnp.full((16,), -jnp.inf, jnp.float32)
        zero = jnp.zeros((16,), jnp.float32)

        dnums = lax.GatherDimensionNumbers(offset_dims=(),
                                           collapsed_slice_dims=(0,),
                                           start_index_map=(0,))

        def shuffle(v, idx):
            return lax.gather(v, idx[:, None], dnums, (1,),
                              mode=lax.GatherScatterMode.PROMISE_IN_BOUNDS)

        def lane_max(v):
            for k in (1, 2, 4, 8):
                v = jnp.maximum(v, shuffle(v, lane ^ k))
            return v

        def lane_min(v):
            for k in (1, 2, 4, 8):
                v = jnp.minimum(v, shuffle(v, lane ^ k))
            return v

        def argmax_first(v, m):
            return lane_min(jnp.where(v == m, lane, N_EXPERTS))

        def group_body(g, _):
            acc = jnp.zeros((16,), jnp.int32)
            for r in range(8):
                i = g * 8 + r
                v = lg_v[i]
                m1 = lane_max(v)
                i1 = argmax_first(v, m1)
                masked = jnp.where(lane == i1, neg_inf, v)
                m2 = lane_max(masked)
                i2 = argmax_first(masked, m2)
                e = jnp.exp(m2 - m1)
                g2 = e / (1.0 + e)
                g1 = 1.0 - g2
                lg_v[i] = jnp.where(lane == i1, g1,
                                    jnp.where(lane == i2, g2, zero))
                acc = jnp.where(lane == 2 * r, i1, acc)
                acc = jnp.where(lane == 2 * r + 1, i2, acc)
            idx_v[pl.ds(g * 16, 16)] = acc
            return 0

        lax.fori_loop(0, groups, group_body, 0)
        pltpu.sync_copy(lg_v, gates_hbm.at[pl.ds(base, rpw)])
        pltpu.sync_copy(idx_v, idx_hbm.at[pl.ds(base * TOPK, rpw * TOPK)])

    gates, idx_flat = route(logits)
    return gates, idx_flat.reshape(rows, TOPK)


@jax.jit
def kernel(x, W, b):
    x = x.astype(jnp.float32)
    b2 = b.reshape(1, N_EXPERTS).astype(jnp.float32)
    logits = _tc_logits(x, W.astype(jnp.float32), b2)
    return _sc_route(logits)


# speedup vs baseline: 1.1103x; 1.1103x over previous
import functools

import jax
import jax.numpy as jnp
from jax import lax
from jax.experimental import pallas as pl
from jax.experimental.pallas import tpu as pltpu
from jax.experimental.pallas import tpu_sc as plsc

BLOCK_ROWS = 2048
N_EXPERTS = 16
TOPK = 2
NC, NS, LANES = 2, 16, 16
NW = NC * NS


def _logits_block(x_ref, w_ref, b_ref, logits_ref):
    logits_ref[...] = jnp.dot(x_ref[...], w_ref[...],
                              preferred_element_type=jnp.float32) + b_ref[...]


def _tc_logits(x, W, b2, chunk, rows):
    blk0 = chunk * (rows // BLOCK_ROWS)
    return pl.pallas_call(
        _logits_block,
        grid=(rows // BLOCK_ROWS,),
        in_specs=[
            pl.BlockSpec((BLOCK_ROWS, x.shape[1]), lambda i: (blk0 + i, 0)),
            pl.BlockSpec((x.shape[1], N_EXPERTS), lambda i: (0, 0)),
            pl.BlockSpec((1, N_EXPERTS), lambda i: (0, 0)),
        ],
        out_specs=pl.BlockSpec((BLOCK_ROWS, N_EXPERTS), lambda i: (i, 0)),
        out_shape=jax.ShapeDtypeStruct((rows, N_EXPERTS), jnp.float32),
    )(x, W, b2)


def _sc_route(logits):
    rows = logits.shape[0]
    rpw = rows // NW
    groups = rpw // 8
    mesh = plsc.VectorSubcoreMesh(core_axis_name="c", subcore_axis_name="s",
                                  num_cores=NC, num_subcores=NS)

    @functools.partial(
        pl.kernel,
        out_type=[
            jax.ShapeDtypeStruct((rows, N_EXPERTS), jnp.float32),
            jax.ShapeDtypeStruct((rows * TOPK,), jnp.int32),
        ],
        mesh=mesh,
        scratch_types=[
            pltpu.VMEM((rpw, N_EXPERTS), jnp.float32),
            pltpu.VMEM((rpw * TOPK,), jnp.int32),
        ],
    )
    def route(logits_hbm, gates_hbm, idx_hbm, lg_v, idx_v):
        wid = lax.axis_index("s") * NC + lax.axis_index("c")
        base = wid * rpw
        pltpu.sync_copy(logits_hbm.at[pl.ds(base, rpw)], lg_v)
        lane = lax.iota(jnp.int32, 16)
        neg_inf = jnp.full((16,), -jnp.inf, jnp.float32)
        zero = jnp.zeros((16,), jnp.float32)

        dnums = lax.GatherDimensionNumbers(offset_dims=(),
                                           collapsed_slice_dims=(0,),
                                           start_index_map=(0,))

        def shuffle(v, idx):
            return lax.gather(v, idx[:, None], dnums, (1,),
                              mode=lax.GatherScatterMode.PROMISE_IN_BOUNDS)

        def lane_max(v):
            for k in (1, 2, 4, 8):
                v = jnp.maximum(v, shuffle(v, lane ^ k))
            return v

        def lane_min(v):
            for k in (1, 2, 4, 8):
                v = jnp.minimum(v, shuffle(v, lane ^ k))
            return v

        def argmax_first(v, m):
            return lane_min(jnp.where(v == m, lane, N_EXPERTS))

        def group_body(g, _):
            acc = jnp.zeros((16,), jnp.int32)
            for r in range(8):
                i = g * 8 + r
                v = lg_v[i]
                m1 = lane_max(v)
                i1 = argmax_first(v, m1)
                masked = jnp.where(lane == i1, neg_inf, v)
                m2 = lane_max(masked)
                i2 = argmax_first(masked, m2)
                e = jnp.exp(m2 - m1)
                g2 = e / (1.0 + e)
                g1 = 1.0 - g2
                lg_v[i] = jnp.where(lane == i1, g1,
                                    jnp.where(lane == i2, g2, zero))
                acc = jnp.where(lane == 2 * r, i1, acc)
                acc = jnp.where(lane == 2 * r + 1, i2, acc)
            idx_v[pl.ds(g * 16, 16)] = acc
            return 0

        lax.fori_loop(0, groups, group_body, 0)
        pltpu.sync_copy(lg_v, gates_hbm.at[pl.ds(base, rpw)])
        pltpu.sync_copy(idx_v, idx_hbm.at[pl.ds(base * TOPK, rpw * TOPK)])

    gates, idx_flat = route(logits)
    return gates, idx_flat.reshape(rows, TOPK)


N_CHUNKS = 2


@jax.jit
def kernel(x, W, b):
    x = x.astype(jnp.float32)
    Wf = W.astype(jnp.float32)
    b2 = b.reshape(1, N_EXPERTS).astype(jnp.float32)
    rows = x.shape[0]
    h = rows // N_CHUNKS
    parts = []
    for c in range(N_CHUNKS):
        logits = _tc_logits(x, Wf, b2, c, h)
        parts.append(_sc_route(logits))
    gates = jnp.concatenate([p[0] for p in parts], axis=0)
    idx = jnp.concatenate([p[1] for p in parts], axis=0)
    return gates, idx
